# speedup vs baseline: 6.0481x; 6.0481x over previous
import jax
import jax.numpy as jnp
from jax.experimental import pallas as pl

NUM_EXPERTS = 8
TOP_K = 2
HIDDEN = 768
INTERMEDIATE = 512
TOKENS = 2048


def _moe_body(logits_ref, x_ref, gw_ref, uw_ref, dw_ref, out_ref):
    e = pl.program_id(0)
    x = x_ref[...]
    logits = logits_ref[...]

    m1 = jnp.max(logits, axis=1, keepdims=True)
    iota = jax.lax.broadcasted_iota(jnp.int32, logits.shape, 1)
    big = jnp.int32(NUM_EXPERTS)
    e0 = jnp.min(jnp.where(logits == m1, iota, big), axis=1, keepdims=True)
    masked = jnp.where(iota == e0, -jnp.inf, logits)
    m2 = jnp.max(masked, axis=1, keepdims=True)
    e1 = jnp.min(jnp.where(masked == m2, iota, big), axis=1, keepdims=True)
    w0 = 1.0 / (1.0 + jnp.exp(m2 - m1))
    w1 = 1.0 - w0
    c = jnp.where(e0 == e, w0, 0.0) + jnp.where(e1 == e, w1, 0.0)

    g = jnp.dot(x, gw_ref[0], preferred_element_type=jnp.float32)
    u = jnp.dot(x, uw_ref[0], preferred_element_type=jnp.float32)
    h = (g * jax.nn.sigmoid(g)) * u
    y = jnp.dot(h, dw_ref[0], preferred_element_type=jnp.float32)

    @pl.when(e == 0)
    def _():
        out_ref[...] = c * y

    @pl.when(e > 0)
    def _():
        out_ref[...] += c * y


def kernel(hidden_states, router_logits, gate_w, up_w, down_w):
    return pl.pallas_call(
        _moe_body,
        grid=(NUM_EXPERTS,),
        in_specs=[
            pl.BlockSpec((TOKENS, NUM_EXPERTS), lambda e: (0, 0)),
            pl.BlockSpec((TOKENS, HIDDEN), lambda e: (0, 0)),
            pl.BlockSpec((1, HIDDEN, INTERMEDIATE), lambda e: (e, 0, 0)),
            pl.BlockSpec((1, HIDDEN, INTERMEDIATE), lambda e: (e, 0, 0)),
            pl.BlockSpec((1, INTERMEDIATE, HIDDEN), lambda e: (e, 0, 0)),
        ],
        out_specs=pl.BlockSpec((TOKENS, HIDDEN), lambda e: (0, 0)),
        out_shape=jax.ShapeDtypeStruct((TOKENS, HIDDEN), jnp.float32),
    )(router_logits, hidden_states, gate_w, up_w, down_w)
